# baseline (speedup 1.0000x reference)
import functools

import jax
import jax.numpy as jnp
from jax import lax
from jax.experimental import pallas as pl
from jax.experimental.pallas import tpu as pltpu
from jax.experimental.pallas import tpu_sc as plsc

NC = 2
NS = 16
EB = 128
CH = 24


def _make_segment_sum(n, d, nacc, nb, nc):
    zps = nacc // NS
    per_w = nb * EB

    mesh = plsc.VectorSubcoreMesh(
        core_axis_name="c", subcore_axis_name="s",
        num_cores=nc, num_subcores=NS)

    @functools.partial(
        pl.kernel,
        out_type=jax.ShapeDtypeStruct((nc, nacc, d), jnp.float32),
        mesh=mesh,
        scratch_types=[
            pltpu.VMEM_SHARED((nacc, d), jnp.float32),
            pltpu.VMEM((EB,), jnp.int32),
            pltpu.VMEM((EB,), jnp.int32),
            pltpu.VMEM((EB,), jnp.int32),
            pltpu.VMEM((EB,), jnp.int32),
            pltpu.VMEM((EB,), jnp.int32),
            pltpu.VMEM((EB,), jnp.int32),
            pltpu.VMEM((EB, d), jnp.float32),
            pltpu.VMEM((EB, d), jnp.float32),
            pltpu.SemaphoreType.DMA,
            pltpu.SemaphoreType.DMA,
            pltpu.SemaphoreType.DMA,
        ],
    )
    def seg_sum(h_hbm, src_hbm, dst_hbm, zero_hbm, out_hbm,
                acc, src0, src1, dst0, dst1, dst2, dst3, rows0, rows1,
                gsem, isem, asem):
        c = lax.axis_index("c")
        s = lax.axis_index("s")
        wid = c * NS + s
        base = wid * per_w

        pltpu.sync_copy(zero_hbm, acc.at[pl.ds(s * zps, zps)])
        plsc.subcore_barrier()

        src = (src0, src1)
        dst = (dst0, dst1, dst2, dst3)
        rows = (rows0, rows1)

        def idx_descs(jj, k2, k4):
            return (pltpu.make_async_copy(
                        src_hbm.at[pl.ds(base + jj * EB, EB)], src[k2], isem),
                    pltpu.make_async_copy(
                        dst_hbm.at[pl.ds(base + jj * EB, EB)], dst[k4], isem))

        def load_and_gather(jj, k2, k4):
            a, b = idx_descs(jj, k2, k4)
            a.start()
            b.start()
            a.wait()
            b.wait()
            pltpu.async_copy(h_hbm.at[src[k2]], rows[k2], gsem)

        load_and_gather(0, 0, 0)
        load_and_gather(1, 1, 1)

        def body(i, carry):
            for k in range(4):
                jj = i * 4 + k
                p = k % 2
                pltpu.make_async_copy(h_hbm.at[src[p]], rows[p], gsem).wait()
                sc = pltpu.make_async_copy(rows[p], acc.at[dst[k]], asem)
                sc.start(add=True)

                @pl.when(jj + 2 < nb)
                def _(jj=jj, p=p, k=k):
                    for dsc in idx_descs(jj + 2, p, (k + 2) % 4):
                        dsc.start()

                sc.wait()

                @pl.when(jj + 2 < nb)
                def _(jj=jj, p=p, k=k):
                    for dsc in idx_descs(jj + 2, p, (k + 2) % 4):
                        dsc.wait()
                    pltpu.async_copy(h_hbm.at[src[p]], rows[p], gsem)
            return carry

        lax.fori_loop(0, nb // 4, body, 0)
        plsc.subcore_barrier()

        pltpu.sync_copy(acc.at[pl.ds(s * zps, zps)],
                        out_hbm.at[c].at[pl.ds(s * zps, zps)])

    return seg_sum


def _linear_relu(parts, w, b, n, d, blk, nc):
    nbk = n // blk

    def body(*refs):
        p_refs, (w_ref, b_ref, o_ref) = refs[:nc], refs[nc:]
        msgs = p_refs[0][0]
        for pr in p_refs[1:]:
            msgs = msgs + pr[0]
        y = lax.dot_general(msgs, w_ref[...], (((1,), (0,)), ((), ())),
                            preferred_element_type=jnp.float32)
        o_ref[...] = jnp.maximum(y + b_ref[...], 0.0)

    in_specs = [
        pl.BlockSpec((1, blk, d), functools.partial(lambda cc, i: (cc, i, 0), cc))
        for cc in range(nc)
    ] + [
        pl.BlockSpec((d, d), lambda i: (0, 0)),
        pl.BlockSpec((1, d), lambda i: (0, 0)),
    ]
    return pl.pallas_call(
        body,
        grid=(nbk,),
        in_specs=in_specs,
        out_specs=pl.BlockSpec((blk, d), lambda i: (i, 0)),
        out_shape=jax.ShapeDtypeStruct((n, d), jnp.float32),
    )(*([parts] * nc), w, b.reshape(1, d))


def kernel(x, edge_index, W1, b1, W2, b2):
    n, d = x.shape
    e = edge_index.shape[1]

    loop = jnp.arange(n, dtype=jnp.int32)
    src = jnp.concatenate([edge_index[0].astype(jnp.int32), loop])
    dst = jnp.concatenate([edge_index[1].astype(jnp.int32), loop])

    nc = 1
    etot = e + n
    nw = nc * NS
    nb = -(-etot // (nw * EB * 4)) * 4
    epad = nw * nb * EB - etot
    src = jnp.concatenate([src, jnp.zeros((epad,), jnp.int32)])
    dst = jnp.concatenate([dst, jnp.full((epad,), n, jnp.int32)])

    nacc = -(-(n + 1) // (8 * NS)) * (8 * NS)
    zeros = jnp.zeros((nacc // NS, d), jnp.float32)

    seg = _make_segment_sum(n, d, nacc, nb, nc)

    parts1 = seg(x, src, dst, zeros)
    h1 = _linear_relu(parts1, W1, b1, n, d, 1000, nc)
    parts2 = seg(h1, src, dst, zeros)
    h2 = _linear_relu(parts2, W2, b2, n, d, 1000, nc)
    return h2


# speedup vs baseline: 1.5413x; 1.5413x over previous
import functools

import jax
import jax.numpy as jnp
from jax import lax
from jax.experimental import pallas as pl
from jax.experimental.pallas import tpu as pltpu
from jax.experimental.pallas import tpu_sc as plsc

NC = 2
NS = 16
EB = 128
CH = 24


def _make_segment_sum(n, d, nacc, nb, nc):
    zps = nacc // NS
    per_w = nb * EB

    mesh = plsc.VectorSubcoreMesh(
        core_axis_name="c", subcore_axis_name="s",
        num_cores=nc, num_subcores=NS)

    @functools.partial(
        pl.kernel,
        out_type=jax.ShapeDtypeStruct((nc, nacc, d), jnp.float32),
        mesh=mesh,
        scratch_types=[
            pltpu.VMEM_SHARED((nacc, d), jnp.float32),
            pltpu.VMEM((EB,), jnp.int32),
            pltpu.VMEM((EB,), jnp.int32),
            pltpu.VMEM((EB,), jnp.int32),
            pltpu.VMEM((EB,), jnp.int32),
            pltpu.VMEM((EB, d), jnp.float32),
            pltpu.VMEM((EB, d), jnp.float32),
            pltpu.SemaphoreType.DMA,
            pltpu.SemaphoreType.DMA,
        ],
    )
    def seg_sum(h_hbm, src_hbm, dst_hbm, zero_hbm, out_hbm,
                acc, src0, src1, dst0, dst1, rows0, rows1, gsem, isem):
        c = lax.axis_index("c")
        s = lax.axis_index("s")
        wid = c * NS + s
        base = wid * per_w

        pltpu.sync_copy(zero_hbm, acc.at[pl.ds(s * zps, zps)])
        plsc.subcore_barrier()

        def load_and_gather(jj, src_s, dst_s, rows_s):
            a = pltpu.make_async_copy(
                src_hbm.at[pl.ds(base + jj * EB, EB)], src_s, isem)
            b = pltpu.make_async_copy(
                dst_hbm.at[pl.ds(base + jj * EB, EB)], dst_s, isem)
            a.start()
            b.start()
            a.wait()
            b.wait()
            pltpu.async_copy(h_hbm.at[src_s], rows_s, gsem)

        slots = ((src0, dst0, rows0), (src1, dst1, rows1))
        load_and_gather(0, *slots[0])
        load_and_gather(1, *slots[1])

        def body(i, carry):
            for k, (src_s, dst_s, rows_s) in enumerate(slots):
                jj = i * 2 + k
                pltpu.make_async_copy(h_hbm.at[src_s], rows_s, gsem).wait()
                pltpu.sync_copy(rows_s, acc.at[dst_s], add=True)

                @pl.when(jj + 2 < nb)
                def _(jj=jj, src_s=src_s, dst_s=dst_s, rows_s=rows_s):
                    load_and_gather(jj + 2, src_s, dst_s, rows_s)
            return carry

        lax.fori_loop(0, nb // 2, body, 0)
        plsc.subcore_barrier()

        pltpu.sync_copy(acc.at[pl.ds(s * zps, zps)],
                        out_hbm.at[c].at[pl.ds(s * zps, zps)])

    return seg_sum


def _linear_relu(parts, w, b, n, d, blk, nc):
    nbk = n // blk

    def body(*refs):
        p_refs, (w_ref, b_ref, o_ref) = refs[:nc], refs[nc:]
        msgs = p_refs[0][0]
        for pr in p_refs[1:]:
            msgs = msgs + pr[0]
        y = lax.dot_general(msgs, w_ref[...], (((1,), (0,)), ((), ())),
                            preferred_element_type=jnp.float32)
        o_ref[...] = jnp.maximum(y + b_ref[...], 0.0)

    in_specs = [
        pl.BlockSpec((1, blk, d), functools.partial(lambda cc, i: (cc, i, 0), cc))
        for cc in range(nc)
    ] + [
        pl.BlockSpec((d, d), lambda i: (0, 0)),
        pl.BlockSpec((1, d), lambda i: (0, 0)),
    ]
    return pl.pallas_call(
        body,
        grid=(nbk,),
        in_specs=in_specs,
        out_specs=pl.BlockSpec((blk, d), lambda i: (i, 0)),
        out_shape=jax.ShapeDtypeStruct((n, d), jnp.float32),
    )(*([parts] * nc), w, b.reshape(1, d))


def kernel(x, edge_index, W1, b1, W2, b2):
    n, d = x.shape
    e = edge_index.shape[1]

    loop = jnp.arange(n, dtype=jnp.int32)
    src = jnp.concatenate([edge_index[0].astype(jnp.int32), loop])
    dst = jnp.concatenate([edge_index[1].astype(jnp.int32), loop])

    nc = 1
    etot = e + n
    nw = nc * NS
    nb = -(-etot // (nw * EB))
    nb += nb % 2
    epad = nw * nb * EB - etot
    src = jnp.concatenate([src, jnp.zeros((epad,), jnp.int32)])
    dst = jnp.concatenate([dst, jnp.full((epad,), n, jnp.int32)])

    nacc = -(-(n + 1) // (8 * NS)) * (8 * NS)
    zeros = jnp.zeros((nacc // NS, d), jnp.float32)

    seg = _make_segment_sum(n, d, nacc, nb, nc)

    parts1 = seg(x, src, dst, zeros)
    h1 = _linear_relu(parts1, W1, b1, n, d, 1000, nc)
    parts2 = seg(h1, src, dst, zeros)
    h2 = _linear_relu(parts2, W2, b2, n, d, 1000, nc)
    return h2


# speedup vs baseline: 1.5476x; 1.0040x over previous
import functools

import jax
import jax.numpy as jnp
from jax import lax
from jax.experimental import pallas as pl
from jax.experimental.pallas import tpu as pltpu
from jax.experimental.pallas import tpu_sc as plsc

NS = 16
EB = 128


def _make_segment_sum(n, d, nacc, nb, nc):
    zps = nacc // NS
    per_w = nb * EB

    mesh = plsc.VectorSubcoreMesh(
        core_axis_name="c", subcore_axis_name="s",
        num_cores=nc, num_subcores=NS)

    @functools.partial(
        pl.kernel,
        out_type=jax.ShapeDtypeStruct((nc, nacc, d), jnp.float32),
        mesh=mesh,
        scratch_types=[
            pltpu.VMEM_SHARED((nacc, d), jnp.float32),
            pltpu.VMEM((EB,), jnp.int32),
            pltpu.VMEM((EB,), jnp.int32),
            pltpu.VMEM((EB,), jnp.int32),
            pltpu.VMEM((EB,), jnp.int32),
            pltpu.VMEM((EB, d), jnp.float32),
            pltpu.VMEM((EB, d), jnp.float32),
            pltpu.SemaphoreType.DMA,
            pltpu.SemaphoreType.DMA,
        ],
    )
    def seg_sum(h_hbm, src_hbm, dst_hbm, zero_hbm, out_hbm,
                acc, src0, src1, dst0, dst1, rows0, rows1, gsem, isem):
        c = lax.axis_index("c")
        s = lax.axis_index("s")
        wid = c * NS + s
        base = wid * per_w

        pltpu.sync_copy(zero_hbm, acc.at[pl.ds(s * zps, zps)])
        plsc.subcore_barrier()

        def load_and_gather(jj, src_s, dst_s, rows_s):
            a = pltpu.make_async_copy(
                src_hbm.at[pl.ds(base + jj * EB, EB)], src_s, isem)
            b = pltpu.make_async_copy(
                dst_hbm.at[pl.ds(base + jj * EB, EB)], dst_s, isem)
            a.start()
            b.start()
            a.wait()
            b.wait()
            pltpu.async_copy(h_hbm.at[src_s], rows_s, gsem)

        slots = ((src0, dst0, rows0), (src1, dst1, rows1))
        load_and_gather(0, *slots[0])
        load_and_gather(1, *slots[1])

        def body(i, carry):
            for k, (src_s, dst_s, rows_s) in enumerate(slots):
                jj = i * 2 + k
                pltpu.make_async_copy(h_hbm.at[src_s], rows_s, gsem).wait()
                pltpu.sync_copy(rows_s, acc.at[dst_s], add=True)

                @pl.when(jj + 2 < nb)
                def _(jj=jj, src_s=src_s, dst_s=dst_s, rows_s=rows_s):
                    load_and_gather(jj + 2, src_s, dst_s, rows_s)
            return carry

        lax.fori_loop(0, nb // 2, body, 0)
        plsc.subcore_barrier()

        pltpu.sync_copy(acc.at[pl.ds(s * zps, zps)],
                        out_hbm.at[c].at[pl.ds(s * zps, zps)])

    return seg_sum


def _linear_relu(parts, w, b, n, d, blk, nc):
    nbk = n // blk

    def body(*refs):
        p_refs, (w_ref, b_ref, o_ref) = refs[:nc], refs[nc:]
        msgs = p_refs[0][0]
        for pr in p_refs[1:]:
            msgs = msgs + pr[0]
        y = lax.dot_general(msgs, w_ref[...], (((1,), (0,)), ((), ())),
                            preferred_element_type=jnp.float32)
        o_ref[...] = jnp.maximum(y + b_ref[...], 0.0)

    in_specs = [
        pl.BlockSpec((1, blk, d), functools.partial(lambda cc, i: (cc, i, 0), cc))
        for cc in range(nc)
    ] + [
        pl.BlockSpec((d, d), lambda i: (0, 0)),
        pl.BlockSpec((1, d), lambda i: (0, 0)),
    ]
    return pl.pallas_call(
        body,
        grid=(nbk,),
        in_specs=in_specs,
        out_specs=pl.BlockSpec((blk, d), lambda i: (i, 0)),
        out_shape=jax.ShapeDtypeStruct((n, d), jnp.float32),
    )(*([parts] * nc), w, b.reshape(1, d))


def kernel(x, edge_index, W1, b1, W2, b2):
    n, d = x.shape
    e = edge_index.shape[1]

    loop = jnp.arange(n, dtype=jnp.int32)
    src = jnp.concatenate([edge_index[0].astype(jnp.int32), loop])
    dst = jnp.concatenate([edge_index[1].astype(jnp.int32), loop])

    nc = 1
    etot = e + n
    nw = nc * NS
    nb = -(-etot // (nw * EB))
    nb += nb % 2
    epad = nw * nb * EB - etot
    src = jnp.concatenate([src, jnp.zeros((epad,), jnp.int32)])
    dst = jnp.concatenate([dst, jnp.full((epad,), n, jnp.int32)])

    nacc = -(-(n + 1) // (8 * NS)) * (8 * NS)
    zeros = jnp.zeros((nacc // NS, d), jnp.float32)

    seg = _make_segment_sum(n, d, nacc, nb, nc)

    parts1 = seg(x, src, dst, zeros)
    h1 = _linear_relu(parts1, W1, b1, n, d, 1000, nc)
    parts2 = seg(h1, src, dst, zeros)
    h2 = _linear_relu(parts2, W2, b2, n, d, 1000, nc)
    return h2
